# baseline (speedup 1.0000x reference)
import functools

import jax
import jax.numpy as jnp
from jax import lax
from jax.experimental import pallas as pl
from jax.experimental.pallas import tpu as pltpu
from jax.experimental.pallas import tpu_sc as plsc

_NC = 2
_NS = 16
_NW = _NC * _NS

_B = 16384
_D = 64
_BPW = _B // _NW
_CHUNK = 128


_NSEM = 8


def _sc_gather(user_ids, item_ids, user_table, item_table):
  mesh = plsc.VectorSubcoreMesh(
      core_axis_name="c", subcore_axis_name="s",
      num_cores=_NC, num_subcores=_NS)

  @functools.partial(
      pl.kernel,
      out_type=[
          jax.ShapeDtypeStruct((_B, _D), jnp.float32),
          jax.ShapeDtypeStruct((_B, _D), jnp.float32),
      ],
      mesh=mesh,
      scratch_types=[
          pltpu.VMEM((_BPW, _D), jnp.float32),
          pltpu.VMEM_SHARED((_NS, _BPW, _D), jnp.float32),
          pltpu.VMEM((_BPW,), jnp.int32),
          pltpu.VMEM((_BPW,), jnp.int32),
          [pltpu.SemaphoreType.DMA] * _NSEM,
          [pltpu.SemaphoreType.DMA] * _NSEM,
      ],
  )
  def gather_kernel(uid_hbm, iid_hbm, ut_hbm, it_hbm, ue_out, ie_out,
                    urows_v, irows_sh, uidx_s, iidx_s, ssems, dsems):
    cid = lax.axis_index("c")
    sid = lax.axis_index("s")
    wid = sid * _NC + cid
    base = wid * _BPW
    pltpu.sync_copy(uid_hbm.at[pl.ds(base, _BPW)], uidx_s)
    pltpu.sync_copy(iid_hbm.at[pl.ds(base, _BPW)], iidx_s)

    def row_dma(g, _):
      uvec = uidx_s[pl.ds(g * 16, 16)]
      ivec = iidx_s[pl.ds(g * 16, 16)]
      for j in range(16):
        slot = g * 16 + j
        pltpu.async_copy(ut_hbm.at[uvec[j]], urows_v.at[slot],
                         ssems[j % _NSEM])
        pltpu.async_copy(it_hbm.at[ivec[j]], irows_sh.at[sid, slot],
                         dsems[j % _NSEM])
      return 0

    lax.fori_loop(0, _BPW // 16, row_dma, 0)
    for k in range(_NSEM):
      pltpu.make_async_copy(
          ut_hbm.at[pl.ds(0, _BPW // _NSEM)],
          urows_v.at[pl.ds(0, _BPW // _NSEM)],
          ssems[k],
      ).wait()
      pltpu.make_async_copy(
          it_hbm.at[pl.ds(0, _BPW // _NSEM)],
          irows_sh.at[sid, pl.ds(0, _BPW // _NSEM)],
          dsems[k],
      ).wait()
    pltpu.sync_copy(urows_v, ue_out.at[pl.ds(base, _BPW)])
    pltpu.sync_copy(irows_sh.at[sid], ie_out.at[pl.ds(base, _BPW)])

  return gather_kernel(user_ids, item_ids, user_table, item_table)


def _mlp_body(ue_ref, ie_ref, w0a_ref, w0b_ref, b0_ref, w1_ref, b1_ref,
              w2_ref, b2_ref, wout_ref, bout_ref, out_ref):
  h = jnp.dot(ue_ref[...], w0a_ref[...], preferred_element_type=jnp.float32)
  h += jnp.dot(ie_ref[...], w0b_ref[...], preferred_element_type=jnp.float32)
  h = jnp.maximum(h + b0_ref[...], 0.0)
  h = jnp.dot(h, w1_ref[...], preferred_element_type=jnp.float32)
  h = jnp.maximum(h + b1_ref[...], 0.0)
  h = jnp.dot(h, w2_ref[...], preferred_element_type=jnp.float32)
  h = jnp.maximum(h + b2_ref[...], 0.0)
  p = jnp.dot(h, wout_ref[...], preferred_element_type=jnp.float32)
  out_ref[...] = jax.nn.sigmoid(p + bout_ref[...])


_MLP_BLK = 2048


def _tc_mlp(ue, ie, w0a, w0b, b0, w1, b1, w2, b2, wout, bout):
  grid = _B // _MLP_BLK
  full = lambda shape: pl.BlockSpec(shape, lambda i: (0,) * len(shape))
  return pl.pallas_call(
      _mlp_body,
      grid=(grid,),
      in_specs=[
          pl.BlockSpec((_MLP_BLK, _D), lambda i: (i, 0)),
          pl.BlockSpec((_MLP_BLK, _D), lambda i: (i, 0)),
          full(w0a.shape), full(w0b.shape), full(b0.shape),
          full(w1.shape), full(b1.shape),
          full(w2.shape), full(b2.shape),
          full(wout.shape), full(bout.shape),
      ],
      out_specs=pl.BlockSpec((_MLP_BLK, 1), lambda i: (i, 0)),
      out_shape=jax.ShapeDtypeStruct((_B, 1), jnp.float32),
  )(ue, ie, w0a, w0b, b0, w1, b1, w2, b2, wout, bout)


def kernel(user_ids, item_ids, user_table, item_table,
           W0, b0, W1, b1, W2, b2, Wout, bout):
  user_ids = user_ids.astype(jnp.int32)
  item_ids = item_ids.astype(jnp.int32)
  ue, ie = _sc_gather(user_ids, item_ids, user_table, item_table)
  w0a = W0[:_D]
  w0b = W0[_D:]
  return _tc_mlp(ue, ie, w0a, w0b,
                 b0.reshape(1, -1), W1, b1.reshape(1, -1),
                 W2, b2.reshape(1, -1), Wout, bout.reshape(1, 1))


# speedup vs baseline: 1.2778x; 1.2778x over previous
import functools

import jax
import jax.numpy as jnp
from jax import lax
from jax.experimental import pallas as pl
from jax.experimental.pallas import tpu as pltpu
from jax.experimental.pallas import tpu_sc as plsc

_NC = 2
_NS = 16
_NW = _NC * _NS

_B = 16384
_D = 64
_B_SC = 8192
_B_TC = _B - _B_SC
_BPW = _B_SC // _NW
_NSEM = 8


def _sc_gather(user_ids, item_ids, user_table, item_table):
  mesh = plsc.VectorSubcoreMesh(
      core_axis_name="c", subcore_axis_name="s",
      num_cores=_NC, num_subcores=_NS)

  @functools.partial(
      pl.kernel,
      out_type=[
          jax.ShapeDtypeStruct((_B_SC, _D), jnp.float32),
          jax.ShapeDtypeStruct((_B_SC, _D), jnp.float32),
      ],
      mesh=mesh,
      scratch_types=[
          pltpu.VMEM((_BPW, _D), jnp.float32),
          pltpu.VMEM((_BPW, _D), jnp.float32),
          pltpu.VMEM((_BPW,), jnp.int32),
          pltpu.VMEM((_BPW,), jnp.int32),
          [pltpu.SemaphoreType.DMA] * _NSEM,
      ],
  )
  def gather_kernel(uid_hbm, iid_hbm, ut_hbm, it_hbm, ue_out, ie_out,
                    urows_v, irows_v, uidx_s, iidx_s, sems):
    wid = lax.axis_index("s") * _NC + lax.axis_index("c")
    base = wid * _BPW
    pltpu.sync_copy(uid_hbm.at[pl.ds(base, _BPW)], uidx_s)
    pltpu.sync_copy(iid_hbm.at[pl.ds(base, _BPW)], iidx_s)

    def row_dma(g, _):
      uvec = uidx_s[pl.ds(g * 16, 16)]
      ivec = iidx_s[pl.ds(g * 16, 16)]
      for j in range(16):
        sem = sems[(2 * j) % _NSEM]
        sem2 = sems[(2 * j + 1) % _NSEM]
        pltpu.async_copy(ut_hbm.at[uvec[j]], urows_v.at[g * 16 + j], sem)
        pltpu.async_copy(it_hbm.at[ivec[j]], irows_v.at[g * 16 + j], sem2)
      return 0

    lax.fori_loop(0, _BPW // 16, row_dma, 0)
    for k in range(_NSEM):
      pltpu.make_async_copy(
          ut_hbm.at[pl.ds(0, 2 * _BPW // _NSEM)],
          urows_v.at[pl.ds(0, 2 * _BPW // _NSEM)],
          sems[k],
      ).wait()
    pltpu.sync_copy(urows_v, ue_out.at[pl.ds(base, _BPW)])
    pltpu.sync_copy(irows_v, ie_out.at[pl.ds(base, _BPW)])

  return gather_kernel(user_ids, item_ids, user_table, item_table)


def _tc_gather_body(uidx_s, iidx_s, ut_hbm, it_hbm, ue_ref, ie_ref,
                    usem, isem):
  def row_dma(b, _):
    pltpu.make_async_copy(ut_hbm.at[uidx_s[b]], ue_ref.at[b], usem).start()
    pltpu.make_async_copy(it_hbm.at[iidx_s[b]], ie_ref.at[b], isem).start()
    return 0

  lax.fori_loop(0, _B_TC, row_dma, 0, unroll=8)
  pltpu.make_async_copy(ut_hbm.at[pl.ds(0, _B_TC)], ue_ref, usem).wait()
  pltpu.make_async_copy(it_hbm.at[pl.ds(0, _B_TC)], ie_ref, isem).wait()


def _tc_gather(uids_tail, iids_tail, user_table, item_table):
  return pl.pallas_call(
      _tc_gather_body,
      in_specs=[
          pl.BlockSpec(memory_space=pltpu.SMEM),
          pl.BlockSpec(memory_space=pltpu.SMEM),
          pl.BlockSpec(memory_space=pltpu.HBM),
          pl.BlockSpec(memory_space=pltpu.HBM),
      ],
      out_specs=[
          pl.BlockSpec(memory_space=pltpu.VMEM),
          pl.BlockSpec(memory_space=pltpu.VMEM),
      ],
      out_shape=[
          jax.ShapeDtypeStruct((_B_TC, _D), jnp.float32),
          jax.ShapeDtypeStruct((_B_TC, _D), jnp.float32),
      ],
      scratch_shapes=[pltpu.SemaphoreType.DMA, pltpu.SemaphoreType.DMA],
  )(uids_tail, iids_tail, user_table, item_table)


def _mlp_body(ue_ref, ie_ref, w0a_ref, w0b_ref, b0_ref, w1_ref, b1_ref,
              w2_ref, b2_ref, wout_ref, bout_ref, out_ref):
  h = jnp.dot(ue_ref[...], w0a_ref[...], preferred_element_type=jnp.float32)
  h += jnp.dot(ie_ref[...], w0b_ref[...], preferred_element_type=jnp.float32)
  h = jnp.maximum(h + b0_ref[...], 0.0)
  h = jnp.dot(h, w1_ref[...], preferred_element_type=jnp.float32)
  h = jnp.maximum(h + b1_ref[...], 0.0)
  h = jnp.dot(h, w2_ref[...], preferred_element_type=jnp.float32)
  h = jnp.maximum(h + b2_ref[...], 0.0)
  p = jnp.dot(h, wout_ref[...], preferred_element_type=jnp.float32)
  out_ref[...] = jax.nn.sigmoid(p + bout_ref[...])


_MLP_BLK = 2048


def _tc_mlp(ue, ie, w0a, w0b, b0, w1, b1, w2, b2, wout, bout):
  grid = _B // _MLP_BLK
  full = lambda shape: pl.BlockSpec(shape, lambda i: (0,) * len(shape))
  return pl.pallas_call(
      _mlp_body,
      grid=(grid,),
      in_specs=[
          pl.BlockSpec((_MLP_BLK, _D), lambda i: (i, 0)),
          pl.BlockSpec((_MLP_BLK, _D), lambda i: (i, 0)),
          full(w0a.shape), full(w0b.shape), full(b0.shape),
          full(w1.shape), full(b1.shape),
          full(w2.shape), full(b2.shape),
          full(wout.shape), full(bout.shape),
      ],
      out_specs=pl.BlockSpec((_MLP_BLK, 1), lambda i: (i, 0)),
      out_shape=jax.ShapeDtypeStruct((_B, 1), jnp.float32),
  )(ue, ie, w0a, w0b, b0, w1, b1, w2, b2, wout, bout)


def kernel(user_ids, item_ids, user_table, item_table,
           W0, b0, W1, b1, W2, b2, Wout, bout):
  user_ids = user_ids.astype(jnp.int32)
  item_ids = item_ids.astype(jnp.int32)
  ue_sc, ie_sc = _sc_gather(user_ids[:_B_SC], item_ids[:_B_SC],
                            user_table, item_table)
  ue_tc, ie_tc = _tc_gather(user_ids[_B_SC:], item_ids[_B_SC:],
                            user_table, item_table)
  ue = jnp.concatenate([ue_sc, ue_tc], axis=0)
  ie = jnp.concatenate([ie_sc, ie_tc], axis=0)
  w0a = W0[:_D]
  w0b = W0[_D:]
  return _tc_mlp(ue, ie, w0a, w0b,
                 b0.reshape(1, -1), W1, b1.reshape(1, -1),
                 W2, b2.reshape(1, -1), Wout, bout.reshape(1, 1))


# speedup vs baseline: 1.3227x; 1.0351x over previous
import functools

import jax
import jax.numpy as jnp
from jax import lax
from jax.experimental import pallas as pl
from jax.experimental.pallas import tpu as pltpu
from jax.experimental.pallas import tpu_sc as plsc

_NC = 2
_NS = 16
_NW = _NC * _NS

_B = 16384
_D = 64
_B_SC = 12288
_B_TC = _B - _B_SC
_BPW = _B_SC // _NW
_NSEM = 8


def _sc_gather(user_ids, item_ids, user_table, item_table):
  mesh = plsc.VectorSubcoreMesh(
      core_axis_name="c", subcore_axis_name="s",
      num_cores=_NC, num_subcores=_NS)

  @functools.partial(
      pl.kernel,
      out_type=[
          jax.ShapeDtypeStruct((_B_SC, _D), jnp.float32),
          jax.ShapeDtypeStruct((_B_SC, _D), jnp.float32),
      ],
      mesh=mesh,
      scratch_types=[
          pltpu.VMEM((_BPW, _D), jnp.float32),
          pltpu.VMEM((_BPW, _D), jnp.float32),
          pltpu.VMEM((_BPW,), jnp.int32),
          pltpu.VMEM((_BPW,), jnp.int32),
          [pltpu.SemaphoreType.DMA] * _NSEM,
      ],
  )
  def gather_kernel(uid_hbm, iid_hbm, ut_hbm, it_hbm, ue_out, ie_out,
                    urows_v, irows_v, uidx_s, iidx_s, sems):
    wid = lax.axis_index("s") * _NC + lax.axis_index("c")
    base = wid * _BPW
    pltpu.sync_copy(uid_hbm.at[pl.ds(base, _BPW)], uidx_s)
    pltpu.sync_copy(iid_hbm.at[pl.ds(base, _BPW)], iidx_s)

    def row_dma(g, _):
      uvec = uidx_s[pl.ds(g * 16, 16)]
      ivec = iidx_s[pl.ds(g * 16, 16)]
      for j in range(16):
        sem = sems[(2 * j) % _NSEM]
        sem2 = sems[(2 * j + 1) % _NSEM]
        pltpu.async_copy(ut_hbm.at[uvec[j]], urows_v.at[g * 16 + j], sem)
        pltpu.async_copy(it_hbm.at[ivec[j]], irows_v.at[g * 16 + j], sem2)
      return 0

    lax.fori_loop(0, _BPW // 16, row_dma, 0)
    for k in range(_NSEM):
      pltpu.make_async_copy(
          ut_hbm.at[pl.ds(0, 2 * _BPW // _NSEM)],
          urows_v.at[pl.ds(0, 2 * _BPW // _NSEM)],
          sems[k],
      ).wait()
    pltpu.sync_copy(urows_v, ue_out.at[pl.ds(base, _BPW)])
    pltpu.sync_copy(irows_v, ie_out.at[pl.ds(base, _BPW)])

  return gather_kernel(user_ids, item_ids, user_table, item_table)


def _tc_gather_body(uidx_s, iidx_s, ut_hbm, it_hbm, ue_ref, ie_ref, sems):
  def row_dma(g, _):
    for j in range(_NSEM // 2):
      b = g * (_NSEM // 2) + j
      pltpu.make_async_copy(
          ut_hbm.at[uidx_s[b]], ue_ref.at[b], sems[2 * j]).start()
      pltpu.make_async_copy(
          it_hbm.at[iidx_s[b]], ie_ref.at[b], sems[2 * j + 1]).start()
    return 0

  lax.fori_loop(0, _B_TC // (_NSEM // 2), row_dma, 0, unroll=4)
  for k in range(_NSEM // 2):
    pltpu.make_async_copy(
        ut_hbm.at[pl.ds(0, _B_TC // (_NSEM // 2))],
        ue_ref.at[pl.ds(0, _B_TC // (_NSEM // 2))], sems[2 * k]).wait()
    pltpu.make_async_copy(
        it_hbm.at[pl.ds(0, _B_TC // (_NSEM // 2))],
        ie_ref.at[pl.ds(0, _B_TC // (_NSEM // 2))], sems[2 * k + 1]).wait()


def _tc_gather(uids_tail, iids_tail, user_table, item_table):
  return pl.pallas_call(
      _tc_gather_body,
      in_specs=[
          pl.BlockSpec(memory_space=pltpu.SMEM),
          pl.BlockSpec(memory_space=pltpu.SMEM),
          pl.BlockSpec(memory_space=pltpu.HBM),
          pl.BlockSpec(memory_space=pltpu.HBM),
      ],
      out_specs=[
          pl.BlockSpec(memory_space=pltpu.VMEM),
          pl.BlockSpec(memory_space=pltpu.VMEM),
      ],
      out_shape=[
          jax.ShapeDtypeStruct((_B_TC, _D), jnp.float32),
          jax.ShapeDtypeStruct((_B_TC, _D), jnp.float32),
      ],
      scratch_shapes=[[pltpu.SemaphoreType.DMA] * _NSEM],
  )(uids_tail, iids_tail, user_table, item_table)


def _mlp_body(ue_ref, ie_ref, w0a_ref, w0b_ref, b0_ref, w1_ref, b1_ref,
              w2_ref, b2_ref, wout_ref, bout_ref, out_ref):
  h = jnp.dot(ue_ref[...], w0a_ref[...], preferred_element_type=jnp.float32)
  h += jnp.dot(ie_ref[...], w0b_ref[...], preferred_element_type=jnp.float32)
  h = jnp.maximum(h + b0_ref[...], 0.0)
  h = jnp.dot(h, w1_ref[...], preferred_element_type=jnp.float32)
  h = jnp.maximum(h + b1_ref[...], 0.0)
  h = jnp.dot(h, w2_ref[...], preferred_element_type=jnp.float32)
  h = jnp.maximum(h + b2_ref[...], 0.0)
  p = jnp.dot(h, wout_ref[...], preferred_element_type=jnp.float32)
  out_ref[...] = jax.nn.sigmoid(p + bout_ref[...])


_MLP_BLK = 2048


def _tc_mlp(ue, ie, w0a, w0b, b0, w1, b1, w2, b2, wout, bout):
  grid = _B // _MLP_BLK
  full = lambda shape: pl.BlockSpec(shape, lambda i: (0,) * len(shape))
  return pl.pallas_call(
      _mlp_body,
      grid=(grid,),
      in_specs=[
          pl.BlockSpec((_MLP_BLK, _D), lambda i: (i, 0)),
          pl.BlockSpec((_MLP_BLK, _D), lambda i: (i, 0)),
          full(w0a.shape), full(w0b.shape), full(b0.shape),
          full(w1.shape), full(b1.shape),
          full(w2.shape), full(b2.shape),
          full(wout.shape), full(bout.shape),
      ],
      out_specs=pl.BlockSpec((_MLP_BLK, 1), lambda i: (i, 0)),
      out_shape=jax.ShapeDtypeStruct((_B, 1), jnp.float32),
  )(ue, ie, w0a, w0b, b0, w1, b1, w2, b2, wout, bout)


def kernel(user_ids, item_ids, user_table, item_table,
           W0, b0, W1, b1, W2, b2, Wout, bout):
  user_ids = user_ids.astype(jnp.int32)
  item_ids = item_ids.astype(jnp.int32)
  ue_sc, ie_sc = _sc_gather(user_ids[:_B_SC], item_ids[:_B_SC],
                            user_table, item_table)
  ue_tc, ie_tc = _tc_gather(user_ids[_B_SC:], item_ids[_B_SC:],
                            user_table, item_table)
  ue = jnp.concatenate([ue_sc, ue_tc], axis=0)
  ie = jnp.concatenate([ie_sc, ie_tc], axis=0)
  w0a = W0[:_D]
  w0b = W0[_D:]
  return _tc_mlp(ue, ie, w0a, w0b,
                 b0.reshape(1, -1), W1, b1.reshape(1, -1),
                 W2, b2.reshape(1, -1), Wout, bout.reshape(1, 1))


# speedup vs baseline: 1.3970x; 1.0562x over previous
import functools

import jax
import jax.numpy as jnp
from jax import lax
from jax.experimental import pallas as pl
from jax.experimental.pallas import tpu as pltpu
from jax.experimental.pallas import tpu_sc as plsc

_NC = 2
_NS = 16
_NW = _NC * _NS

_B = 16384
_D = 64
_B_SC = _B
_B_TC = _B - _B_SC
_BPW = _B_SC // _NW
_CHUNK = 256
_NSEM = 8


def _sc_gather(user_ids, item_ids, user_table, item_table):
  mesh = plsc.VectorSubcoreMesh(
      core_axis_name="c", subcore_axis_name="s",
      num_cores=_NC, num_subcores=_NS)

  @functools.partial(
      pl.kernel,
      out_type=[
          jax.ShapeDtypeStruct((_B_SC, _D), jnp.float32),
          jax.ShapeDtypeStruct((_B_SC, _D), jnp.float32),
      ],
      mesh=mesh,
      scratch_types=[
          pltpu.VMEM((_CHUNK, _D), jnp.float32),
          pltpu.VMEM((_CHUNK, _D), jnp.float32),
          pltpu.VMEM((_BPW,), jnp.int32),
          pltpu.VMEM((_BPW,), jnp.int32),
          [pltpu.SemaphoreType.DMA] * _NSEM,
      ],
  )
  def gather_kernel(uid_hbm, iid_hbm, ut_hbm, it_hbm, ue_out, ie_out,
                    urows_v, irows_v, uidx_s, iidx_s, sems):
    wid = lax.axis_index("s") * _NC + lax.axis_index("c")
    base = wid * _BPW
    pltpu.sync_copy(uid_hbm.at[pl.ds(base, _BPW)], uidx_s)
    pltpu.sync_copy(iid_hbm.at[pl.ds(base, _BPW)], iidx_s)

    for c in range(_BPW // _CHUNK):
      off = c * _CHUNK

      def row_dma(g, _):
        uvec = uidx_s[pl.ds(off + g * 16, 16)]
        ivec = iidx_s[pl.ds(off + g * 16, 16)]
        for j in range(16):
          sem = sems[(2 * j) % _NSEM]
          sem2 = sems[(2 * j + 1) % _NSEM]
          pltpu.async_copy(ut_hbm.at[uvec[j]], urows_v.at[g * 16 + j], sem)
          pltpu.async_copy(it_hbm.at[ivec[j]], irows_v.at[g * 16 + j], sem2)
        return 0

      lax.fori_loop(0, _CHUNK // 16, row_dma, 0)
      for k in range(_NSEM):
        pltpu.make_async_copy(
            ut_hbm.at[pl.ds(0, 2 * _CHUNK // _NSEM)],
            urows_v.at[pl.ds(0, 2 * _CHUNK // _NSEM)],
            sems[k],
        ).wait()
      pltpu.sync_copy(urows_v, ue_out.at[pl.ds(base + off, _CHUNK)])
      pltpu.sync_copy(irows_v, ie_out.at[pl.ds(base + off, _CHUNK)])

  return gather_kernel(user_ids, item_ids, user_table, item_table)


def _mlp_body(ue_ref, ie_ref, w0a_ref, w0b_ref, b0_ref, w1_ref, b1_ref,
              w2_ref, b2_ref, wout_ref, bout_ref, out_ref):
  h = jnp.dot(ue_ref[...], w0a_ref[...], preferred_element_type=jnp.float32)
  h += jnp.dot(ie_ref[...], w0b_ref[...], preferred_element_type=jnp.float32)
  h = jnp.maximum(h + b0_ref[...], 0.0)
  h = jnp.dot(h, w1_ref[...], preferred_element_type=jnp.float32)
  h = jnp.maximum(h + b1_ref[...], 0.0)
  h = jnp.dot(h, w2_ref[...], preferred_element_type=jnp.float32)
  h = jnp.maximum(h + b2_ref[...], 0.0)
  p = jnp.dot(h, wout_ref[...], preferred_element_type=jnp.float32)
  out_ref[...] = jax.nn.sigmoid(p + bout_ref[...])


_MLP_BLK = 2048


def _tc_mlp(ue, ie, w0a, w0b, b0, w1, b1, w2, b2, wout, bout):
  grid = _B // _MLP_BLK
  full = lambda shape: pl.BlockSpec(shape, lambda i: (0,) * len(shape))
  return pl.pallas_call(
      _mlp_body,
      grid=(grid,),
      in_specs=[
          pl.BlockSpec((_MLP_BLK, _D), lambda i: (i, 0)),
          pl.BlockSpec((_MLP_BLK, _D), lambda i: (i, 0)),
          full(w0a.shape), full(w0b.shape), full(b0.shape),
          full(w1.shape), full(b1.shape),
          full(w2.shape), full(b2.shape),
          full(wout.shape), full(bout.shape),
      ],
      out_specs=pl.BlockSpec((_MLP_BLK, 1), lambda i: (i, 0)),
      out_shape=jax.ShapeDtypeStruct((_B, 1), jnp.float32),
  )(ue, ie, w0a, w0b, b0, w1, b1, w2, b2, wout, bout)


def kernel(user_ids, item_ids, user_table, item_table,
           W0, b0, W1, b1, W2, b2, Wout, bout):
  user_ids = user_ids.astype(jnp.int32)
  item_ids = item_ids.astype(jnp.int32)
  ue, ie = _sc_gather(user_ids, item_ids, user_table, item_table)
  w0a = W0[:_D]
  w0b = W0[_D:]
  return _tc_mlp(ue, ie, w0a, w0b,
                 b0.reshape(1, -1), W1, b1.reshape(1, -1),
                 W2, b2.reshape(1, -1), Wout, bout.reshape(1, 1))
